# baseline (speedup 1.0000x reference)
import jax
import jax.numpy as jnp
from jax.experimental import pallas as pl
from jax.experimental.pallas import tpu as pltpu

_TILE = 1024
_SW = 256


def _gcn_body(a_ref, x_ref, k1_ref, k2_ref, b1_ref, b2_ref, o_ref):
    a = a_ref[...]
    k1 = k1_ref[...]
    k2 = k2_ref[...]
    b1 = b1_ref[...]
    b2 = b2_ref[...]
    bf = jnp.bfloat16
    for i in range(_TILE // _SW):
        x = x_ref[:, i * _SW:(i + 1) * _SW]
        p1 = jnp.dot(a, x, preferred_element_type=jnp.float32, precision=jax.lax.Precision.DEFAULT)
        h1 = jnp.dot(p1.astype(bf), k1, preferred_element_type=jnp.float32, precision=jax.lax.Precision.DEFAULT) + b1
        h1 = jnp.where(h1 >= 0, h1, 0.01 * h1).astype(bf)
        p2 = jnp.dot(a, h1, preferred_element_type=jnp.float32, precision=jax.lax.Precision.DEFAULT)
        h2 = jnp.dot(p2.astype(bf), k2, preferred_element_type=jnp.float32, precision=jax.lax.Precision.DEFAULT) + b2
        o_ref[:, i * _SW:(i + 1) * _SW] = jnp.where(h2 >= 0, h2, 0.01 * h2)


def kernel(X, A, W1, b1, W2, b2):
    B, N, T, F_in = X.shape
    F_sp = W1.shape[1]
    assert F_in == F_sp, "flattened-column layout assumes F_in == F_sp"
    C = B * T * F_in

    bf = jnp.bfloat16
    Xmat = jnp.transpose(X.astype(bf), (1, 0, 2, 3)).reshape(N, C)
    Abf = A.astype(bf)

    nblk = _SW // F_in
    eye = jnp.eye(nblk, dtype=bf)
    K1 = jnp.kron(eye, W1.astype(bf))
    K2 = jnp.kron(eye, W2.astype(bf))
    b1t = jnp.tile(b1, nblk)[None, :]
    b2t = jnp.tile(b2, nblk)[None, :]

    out = pl.pallas_call(
        _gcn_body,
        grid=(C // _TILE,),
        in_specs=[
            pl.BlockSpec((N, N), lambda j: (0, 0)),
            pl.BlockSpec((N, _TILE), lambda j: (0, j)),
            pl.BlockSpec((_SW, _SW), lambda j: (0, 0)),
            pl.BlockSpec((_SW, _SW), lambda j: (0, 0)),
            pl.BlockSpec((1, _SW), lambda j: (0, 0)),
            pl.BlockSpec((1, _SW), lambda j: (0, 0)),
        ],
        out_specs=pl.BlockSpec((N, _TILE), lambda j: (0, j)),
        out_shape=jax.ShapeDtypeStruct((N, C), jnp.float32),
        compiler_params=pltpu.CompilerParams(
            dimension_semantics=("arbitrary",),
        ),
    )(Abf, Xmat, K1, K2, b1t, b2t)

    return jnp.transpose(out.reshape(N, B, T, F_sp), (1, 0, 2, 3))


# speedup vs baseline: 1.4492x; 1.4492x over previous
import jax
import jax.numpy as jnp
from jax.experimental import pallas as pl
from jax.experimental.pallas import tpu as pltpu

_TILE = 1024
_KW = 128


def _lrelu(v):
    return jnp.maximum(v, 0.01 * v)


def _gcn_body(a_ref, x_ref, k1_ref, k2_ref, b1_ref, b2_ref, o_ref):
    a = a_ref[...]
    k1 = k1_ref[...]
    k2 = k2_ref[...]
    b1 = b1_ref[...]
    b2 = b2_ref[...]
    f32 = jnp.float32
    p1 = jnp.dot(a, x_ref[...], preferred_element_type=f32)
    hs = []
    for c in range(_TILE // _KW):
        h = jnp.dot(p1[:, c * _KW:(c + 1) * _KW], k1, preferred_element_type=f32)
        hs.append(_lrelu(h + b1))
    h1 = jnp.concatenate(hs, axis=1)
    p2 = jnp.dot(a, h1, preferred_element_type=f32)
    for c in range(_TILE // _KW):
        h = jnp.dot(p2[:, c * _KW:(c + 1) * _KW], k2, preferred_element_type=f32)
        o_ref[:, c * _KW:(c + 1) * _KW] = _lrelu(h + b2)


def kernel(X, A, W1, b1, W2, b2):
    B, N, T, F_in = X.shape
    F_sp = W1.shape[1]
    assert F_in == F_sp, "flattened-column layout assumes F_in == F_sp"
    C = B * T * F_in

    Xmat = jnp.transpose(X, (1, 0, 2, 3)).reshape(N, C)

    nblk = _KW // F_in
    eye = jnp.eye(nblk, dtype=X.dtype)
    K1 = jnp.kron(eye, W1)
    K2 = jnp.kron(eye, W2)
    b1t = jnp.tile(b1, nblk)[None, :]
    b2t = jnp.tile(b2, nblk)[None, :]

    out = pl.pallas_call(
        _gcn_body,
        grid=(C // _TILE,),
        in_specs=[
            pl.BlockSpec((N, N), lambda j: (0, 0)),
            pl.BlockSpec((N, _TILE), lambda j: (0, j)),
            pl.BlockSpec((_KW, _KW), lambda j: (0, 0)),
            pl.BlockSpec((_KW, _KW), lambda j: (0, 0)),
            pl.BlockSpec((1, _KW), lambda j: (0, 0)),
            pl.BlockSpec((1, _KW), lambda j: (0, 0)),
        ],
        out_specs=pl.BlockSpec((N, _TILE), lambda j: (0, j)),
        out_shape=jax.ShapeDtypeStruct((N, C), jnp.float32),
        compiler_params=pltpu.CompilerParams(
            dimension_semantics=("arbitrary",),
        ),
    )(A, Xmat, K1, K2, b1t, b2t)

    return jnp.transpose(out.reshape(N, B, T, F_sp), (1, 0, 2, 3))
